# baseline (speedup 1.0000x reference)
import functools

import jax
import jax.numpy as jnp
from jax import lax
from jax.experimental import pallas as pl
from jax.experimental.pallas import tpu as pltpu
from jax.experimental.pallas import tpu_sc as plsc

N = 10000
E = 160000
D_IN = 256
D_H = 512

NC = 2
NS = 16
EB = 128

EPT = E // NS
CHB = 16
NCHK = 5
NBLK = CHB * NCHK
EPTP = NBLK * EB
ZROWS = 632
ACC_ROWS = NS * ZROWS
DEG_PT = 640
DEG_P = NS * DEG_PT

_mesh = lambda: plsc.VectorSubcoreMesh(core_axis_name="c", subcore_axis_name="s")


def _deg_body(dstp, ones_hbm, zeros_hbm, out, didx_all, ones_v, acc, sem):
  c = lax.axis_index("c")
  s = lax.axis_index("s")
  pltpu.sync_copy(ones_hbm, ones_v)
  pltpu.sync_copy(zeros_hbm, acc.at[pl.ds(s * DEG_PT, DEG_PT)])
  pltpu.sync_copy(dstp.at[s], didx_all)
  plsc.subcore_barrier()

  def step(b, carry):
    pltpu.async_copy(ones_v, acc.at[didx_all.at[b]], sem, add=True).wait()
    return carry

  half = NBLK // 2 + 1

  @pl.when(c == 0)
  def _():
    lax.fori_loop(0, half, step, 0)

  @pl.when(c == 1)
  def _():
    lax.fori_loop(half, NBLK, step, 0)

  plsc.subcore_barrier()
  pltpu.sync_copy(acc.at[pl.ds(s * DEG_PT, DEG_PT)],
                  out.at[c, 0, pl.ds(s * DEG_PT, DEG_PT)])


_deg_kernel = functools.partial(
    pl.kernel,
    out_type=jax.ShapeDtypeStruct((NC, 1, DEG_P), jnp.float32),
    mesh=_mesh(),
    scratch_types=[
        pltpu.VMEM((NBLK, EB), jnp.int32),
        pltpu.VMEM((EB,), jnp.float32),
        pltpu.VMEM_SHARED((DEG_P,), jnp.float32),
        pltpu.SemaphoreType.DMA,
    ],
)(_deg_body)


def _make_agg(nch):
  per_core = nch // NC

  def body(srcp, dstp, zeros_hbm, *rest):
    chunks = rest[:nch]
    outs = rest[nch:2 * nch]
    sidx_st, didx_st, buf0, buf1, acc, gsem, ssem = rest[2 * nch:]
    bufs = (buf0, buf1)
    c = lax.axis_index("c")
    s = lax.axis_index("s")

    def chunk_pass(table, out):
      pltpu.sync_copy(zeros_hbm, acc.at[pl.ds(s * ZROWS, ZROWS), :])
      plsc.subcore_barrier()

      def g_desc(j, buf):
        return pltpu.make_async_copy(table.at[sidx_st.at[j]], buf, gsem)

      def s_desc(j, buf):
        return pltpu.make_async_copy(buf, acc.at[didx_st.at[j]], ssem)

      def stage_chunk(ci, carry):
        pltpu.sync_copy(srcp.at[s, pl.ds(ci * CHB, CHB)], sidx_st)
        pltpu.sync_copy(dstp.at[s, pl.ds(ci * CHB, CHB)], didx_st)
        g_desc(0, bufs[0]).start()
        for j in range(CHB):
          g_desc(j, bufs[j % 2]).wait()
          if j >= 1:
            s_desc(j - 1, bufs[(j - 1) % 2]).wait()
          if j + 1 < CHB:
            g_desc(j + 1, bufs[(j + 1) % 2]).start()
          s_desc(j, bufs[j % 2]).start(add=True)
        s_desc(CHB - 1, bufs[(CHB - 1) % 2]).wait()
        return carry

      lax.fori_loop(0, NCHK, stage_chunk, 0)
      plsc.subcore_barrier()
      pltpu.sync_copy(acc.at[pl.ds(s * ZROWS, ZROWS), :],
                      out.at[pl.ds(s * ZROWS, ZROWS), :])
      plsc.subcore_barrier()

    for cc in range(NC):
      @pl.when(c == cc)
      def _(cc=cc):
        for j in range(per_core):
          k = cc * per_core + j
          chunk_pass(chunks[k], outs[k])

  return pl.kernel(
      body,
      out_type=[jax.ShapeDtypeStruct((ACC_ROWS, 128), jnp.float32)
                for _ in range(nch)],
      mesh=_mesh(),
      scratch_types=[
          pltpu.VMEM((CHB, EB), jnp.int32),
          pltpu.VMEM((CHB, EB), jnp.int32),
          pltpu.VMEM((EB, 128), jnp.float32),
          pltpu.VMEM((EB, 128), jnp.float32),
          pltpu.VMEM_SHARED((ACC_ROWS, 128), jnp.float32),
          pltpu.SemaphoreType.DMA,
          pltpu.SemaphoreType.DMA,
      ],
  )


_agg2 = _make_agg(2)
_agg4 = _make_agg(4)


RB = 1000
GRID = N // RB

def _row_spec(w):
  return pl.BlockSpec((RB, w), lambda i: (i, 0))

def _full_spec(h, w):
  return pl.BlockSpec((h, w), lambda i: (0, 0))

def _vec_spec(n):
  return pl.BlockSpec((n,), lambda i: (0,))


def _finalize_deg_body(degp_ref, out_ref):
  deg = degp_ref[0:1, :] + degp_ref[1:2, :] + 1.0
  out_ref[...] = lax.rsqrt(deg)


def _finalize_deg(degp):
  return pl.pallas_call(
      _finalize_deg_body,
      out_shape=jax.ShapeDtypeStruct((1, DEG_P), jnp.float32),
  )(degp)


def _prep_body(dinv_ref, x_ref, o0, o1):
  ux = x_ref[...] * dinv_ref[...]
  o0[...] = ux[:, :128]
  o1[...] = ux[:, 128:]


def _prep(dinv, x):
  return pl.pallas_call(
      _prep_body,
      grid=(GRID,),
      in_specs=[_row_spec(1), _row_spec(D_IN)],
      out_specs=[_row_spec(128), _row_spec(128)],
      out_shape=[jax.ShapeDtypeStruct((N, 128), jnp.float32)] * 2,
  )(dinv, x)


def _ln_elu(v, g, b):
  mu = jnp.mean(v, axis=1, keepdims=True)
  d = v - mu
  var = jnp.mean(d * d, axis=1, keepdims=True)
  hn = d * lax.rsqrt(var + 1e-5) * g + b
  return jnp.where(hn > 0, hn, jnp.exp(hn) - 1.0)


def _layer0_body(dinv_ref, W0_ref, b0_ref, g0_ref, be0_ref, W1_ref,
                 a0, a1, u0, u1, *outs):
  dinv = dinv_ref[...]
  agg = jnp.concatenate([a0[...], a1[...]], axis=1)
  u = jnp.concatenate([u0[...], u1[...]], axis=1)
  sx = (agg + u) * dinv
  pre = jnp.dot(sx, W0_ref[...], preferred_element_type=jnp.float32) + b0_ref[...]
  h = _ln_elu(pre, g0_ref[...], be0_ref[...])
  un = jnp.dot(h, W1_ref[...], preferred_element_type=jnp.float32) * dinv
  for k, o in enumerate(outs):
    o[...] = un[:, k * 128:(k + 1) * 128]


def _layer0(dinv, W0, b0, g0, be0, W1, aggs, us):
  return pl.pallas_call(
      _layer0_body,
      grid=(GRID,),
      in_specs=[_row_spec(1), _full_spec(D_IN, D_H), _vec_spec(D_H),
                _vec_spec(D_H), _vec_spec(D_H), _full_spec(D_H, D_H),
                _row_spec(128), _row_spec(128), _row_spec(128), _row_spec(128)],
      out_specs=[_row_spec(128)] * 4,
      out_shape=[jax.ShapeDtypeStruct((N, 128), jnp.float32)] * 4,
  )(dinv, W0, b0, g0, be0, W1, *aggs, *us)


def _mid_body(dinv_ref, b_ref, g_ref, be_ref, W_ref, *rest):
  aggs, us, outs = rest[:4], rest[4:8], rest[8:]
  dinv = dinv_ref[...]
  agg = jnp.concatenate([r[...] for r in aggs], axis=1)
  u = jnp.concatenate([r[...] for r in us], axis=1)
  conv = (agg + u) * dinv + b_ref[...]
  h = _ln_elu(conv, g_ref[...], be_ref[...])
  un = jnp.dot(h, W_ref[...], preferred_element_type=jnp.float32) * dinv
  for k, o in enumerate(outs):
    o[...] = un[:, k * 128:(k + 1) * 128]


def _mid_layer(dinv, b, g, be, W, aggs, us, nout):
  return pl.pallas_call(
      _mid_body,
      grid=(GRID,),
      in_specs=[_row_spec(1), _vec_spec(D_H), _vec_spec(D_H), _vec_spec(D_H),
                _full_spec(D_H, W.shape[1])] + [_row_spec(128)] * 8,
      out_specs=[_row_spec(128)] * nout,
      out_shape=[jax.ShapeDtypeStruct((N, 128), jnp.float32)] * nout,
  )(dinv, b, g, be, W, *aggs, *us)


def _final_body(dinv_ref, b3_ref, x_ref, a0, a1, u0, u1, out_ref):
  agg = jnp.concatenate([a0[...], a1[...]], axis=1)
  u = jnp.concatenate([u0[...], u1[...]], axis=1)
  out_ref[...] = (agg + u) * dinv_ref[...] + b3_ref[...] + x_ref[...]


def _final(dinv, b3, x, aggs, us):
  return pl.pallas_call(
      _final_body,
      grid=(GRID,),
      in_specs=[_row_spec(1), _vec_spec(D_IN), _row_spec(D_IN)]
               + [_row_spec(128)] * 4,
      out_specs=_row_spec(D_IN),
      out_shape=jax.ShapeDtypeStruct((N, D_IN), jnp.float32),
  )(dinv, b3, x, *aggs, *us)


def kernel(x, edge_index, W0, b0, W1, b1, W2, b2, W3, b3,
           g0, be0, g1, be1, g2, be2):
  src = edge_index[0].reshape(NS, EPT)
  dst = edge_index[1].reshape(NS, EPT)
  npad = EPTP - EPT
  pad_src = jnp.broadcast_to((jnp.arange(npad, dtype=jnp.int32) * 521) % N,
                             (NS, npad))
  pad_dst = jnp.broadcast_to(N + (jnp.arange(npad, dtype=jnp.int32) % NS),
                             (NS, npad))
  srcp = jnp.concatenate([src, pad_src], axis=1).reshape(NS, NBLK, EB)
  dstp = jnp.concatenate([dst, pad_dst], axis=1).reshape(NS, NBLK, EB)

  ones_eb = jnp.ones((EB,), jnp.float32)
  zeros_deg = jnp.zeros((DEG_PT,), jnp.float32)
  zeros_acc = jnp.zeros((ZROWS, 128), jnp.float32)

  degp = _deg_kernel(dstp, ones_eb, zeros_deg).reshape(NC, DEG_P)
  dinv_row = _finalize_deg(degp)
  dinv = dinv_row.reshape(DEG_P, 1)[:N]

  ux = _prep(dinv, x)
  aggx = _agg2(srcp, dstp, zeros_acc, *ux)
  u1 = _layer0(dinv, W0, b0, g0, be0, W1, aggx, ux)
  agg1 = _agg4(srcp, dstp, zeros_acc, *u1)
  u2 = _mid_layer(dinv, b1, g1, be1, W2, agg1, u1, 4)
  agg2 = _agg4(srcp, dstp, zeros_acc, *u2)
  u3 = _mid_layer(dinv, b2, g2, be2, W3, agg2, u2, 2)
  agg3 = _agg2(srcp, dstp, zeros_acc, *u3)
  return _final(dinv, b3, x, agg3, u3)


# speedup vs baseline: 1.0373x; 1.0373x over previous
import functools

import jax
import jax.numpy as jnp
from jax import lax
from jax.experimental import pallas as pl
from jax.experimental.pallas import tpu as pltpu
from jax.experimental.pallas import tpu_sc as plsc

N = 10000
E = 160000
D_IN = 256
D_H = 512

NC = 2
NS = 16
EB = 128

EPT = E // NS
CHB = 16
NCHK = 5
NBLK = CHB * NCHK
EPTP = NBLK * EB
ZROWS = 632
ACC_ROWS = NS * ZROWS
DEG_PT = 640
DEG_P = NS * DEG_PT

_mesh = lambda: plsc.VectorSubcoreMesh(core_axis_name="c", subcore_axis_name="s")


def _deg_body(dstp, ones_hbm, zeros_hbm, out, didx_all, ones_v, acc, sem):
  c = lax.axis_index("c")
  s = lax.axis_index("s")
  pltpu.sync_copy(ones_hbm, ones_v)
  pltpu.sync_copy(zeros_hbm, acc.at[pl.ds(s * DEG_PT, DEG_PT)])
  pltpu.sync_copy(dstp.at[s], didx_all)
  plsc.subcore_barrier()

  def step(b, carry):
    pltpu.async_copy(ones_v, acc.at[didx_all.at[b]], sem, add=True).wait()
    return carry

  half = NBLK // 2

  @pl.when(c == 0)
  def _():
    lax.fori_loop(0, half, step, 0)

  @pl.when(c == 1)
  def _():
    lax.fori_loop(half, NBLK, step, 0)

  plsc.subcore_barrier()
  pltpu.sync_copy(acc.at[pl.ds(s * DEG_PT, DEG_PT)],
                  out.at[c, 0, pl.ds(s * DEG_PT, DEG_PT)])


_deg_kernel = functools.partial(
    pl.kernel,
    out_type=jax.ShapeDtypeStruct((NC, 1, DEG_P), jnp.float32),
    mesh=_mesh(),
    scratch_types=[
        pltpu.VMEM((NBLK, EB), jnp.int32),
        pltpu.VMEM((EB,), jnp.float32),
        pltpu.VMEM_SHARED((DEG_P,), jnp.float32),
        pltpu.SemaphoreType.DMA,
    ],
)(_deg_body)


def _make_agg(nch):
  per_core = nch // NC

  def body(srcp, dstp, zeros_hbm, *rest):
    chunks = rest[:nch]
    outs = rest[nch:2 * nch]
    (sidx0, didx0, sidx1, didx1, buf0, buf1,
     acc, gsem, ssem, isem) = rest[2 * nch:]
    bufs = (buf0, buf1)
    idxsets = ((sidx0, didx0), (sidx1, didx1))
    c = lax.axis_index("c")
    s = lax.axis_index("s")

    def stage_descs(ci, iset):
      return (pltpu.make_async_copy(srcp.at[s, pl.ds(ci * CHB, CHB)],
                                    iset[0], isem),
              pltpu.make_async_copy(dstp.at[s, pl.ds(ci * CHB, CHB)],
                                    iset[1], isem))

    def chunk_pass(table, out):
      pltpu.sync_copy(zeros_hbm, acc.at[pl.ds(s * ZROWS, ZROWS), :])
      for d in stage_descs(0, idxsets[0]):
        d.start()
      plsc.subcore_barrier()

      def g_desc(iset, j, buf):
        return pltpu.make_async_copy(table.at[iset[0].at[j]], buf, gsem)

      def s_desc(iset, j, buf):
        return pltpu.make_async_copy(buf, acc.at[iset[1].at[j]], ssem)

      for ci in range(NCHK):
        iset = idxsets[ci % 2]
        for d in stage_descs(ci, iset):
          d.wait()
        if ci + 1 < NCHK:
          for d in stage_descs(ci + 1, idxsets[(ci + 1) % 2]):
            d.start()
        g_desc(iset, 0, bufs[0]).start()

        def pairstep(p, carry, iset=iset):
          for q in range(2):
            j = p * 2 + q

            @pl.when(j >= 1)
            def _():
              s_desc(iset, j - 1, bufs[1 - q]).wait()

            g_desc(iset, j, bufs[q]).wait()

            @pl.when(j + 1 < CHB)
            def _():
              g_desc(iset, j + 1, bufs[1 - q]).start()

            s_desc(iset, j, bufs[q]).start(add=True)
          return carry

        lax.fori_loop(0, CHB // 2, pairstep, 0)
        s_desc(iset, CHB - 1, bufs[(CHB - 1) % 2]).wait()

      plsc.subcore_barrier()
      pltpu.sync_copy(acc.at[pl.ds(s * ZROWS, ZROWS), :],
                      out.at[pl.ds(s * ZROWS, ZROWS), :])
      plsc.subcore_barrier()

    for cc in range(NC):
      @pl.when(c == cc)
      def _(cc=cc):
        for j in range(per_core):
          k = cc * per_core + j
          chunk_pass(chunks[k], outs[k])

  return pl.kernel(
      body,
      out_type=[jax.ShapeDtypeStruct((ACC_ROWS, 128), jnp.float32)
                for _ in range(nch)],
      mesh=_mesh(),
      scratch_types=[
          pltpu.VMEM((CHB, EB), jnp.int32),
          pltpu.VMEM((CHB, EB), jnp.int32),
          pltpu.VMEM((CHB, EB), jnp.int32),
          pltpu.VMEM((CHB, EB), jnp.int32),
          pltpu.VMEM((EB, 128), jnp.float32),
          pltpu.VMEM((EB, 128), jnp.float32),
          pltpu.VMEM_SHARED((ACC_ROWS, 128), jnp.float32),
          pltpu.SemaphoreType.DMA,
          pltpu.SemaphoreType.DMA,
          pltpu.SemaphoreType.DMA,
      ],
  )


_agg2 = _make_agg(2)
_agg4 = _make_agg(4)


RB = 1000
GRID = N // RB

def _row_spec(w):
  return pl.BlockSpec((RB, w), lambda i: (i, 0))

def _full_spec(h, w):
  return pl.BlockSpec((h, w), lambda i: (0, 0))

def _vec_spec(n):
  return pl.BlockSpec((n,), lambda i: (0,))


def _finalize_deg_body(degp_ref, out_ref):
  deg = degp_ref[0:1, :] + degp_ref[1:2, :] + 1.0
  out_ref[...] = lax.rsqrt(deg)


def _finalize_deg(degp):
  return pl.pallas_call(
      _finalize_deg_body,
      out_shape=jax.ShapeDtypeStruct((1, DEG_P), jnp.float32),
  )(degp)


def _prep_body(dinv_ref, x_ref, o0, o1):
  ux = x_ref[...] * dinv_ref[...]
  o0[...] = ux[:, :128]
  o1[...] = ux[:, 128:]


def _prep(dinv, x):
  return pl.pallas_call(
      _prep_body,
      grid=(GRID,),
      in_specs=[_row_spec(1), _row_spec(D_IN)],
      out_specs=[_row_spec(128), _row_spec(128)],
      out_shape=[jax.ShapeDtypeStruct((N, 128), jnp.float32)] * 2,
  )(dinv, x)


def _ln_elu(v, g, b):
  mu = jnp.mean(v, axis=1, keepdims=True)
  d = v - mu
  var = jnp.mean(d * d, axis=1, keepdims=True)
  hn = d * lax.rsqrt(var + 1e-5) * g + b
  return jnp.where(hn > 0, hn, jnp.exp(hn) - 1.0)


def _layer0_body(dinv_ref, W0_ref, b0_ref, g0_ref, be0_ref, W1_ref,
                 a0, a1, u0, u1, *outs):
  dinv = dinv_ref[...]
  agg = jnp.concatenate([a0[...], a1[...]], axis=1)
  u = jnp.concatenate([u0[...], u1[...]], axis=1)
  sx = (agg + u) * dinv
  pre = jnp.dot(sx, W0_ref[...], preferred_element_type=jnp.float32) + b0_ref[...]
  h = _ln_elu(pre, g0_ref[...], be0_ref[...])
  un = jnp.dot(h, W1_ref[...], preferred_element_type=jnp.float32) * dinv
  for k, o in enumerate(outs):
    o[...] = un[:, k * 128:(k + 1) * 128]


def _layer0(dinv, W0, b0, g0, be0, W1, aggs, us):
  return pl.pallas_call(
      _layer0_body,
      grid=(GRID,),
      in_specs=[_row_spec(1), _full_spec(D_IN, D_H), _vec_spec(D_H),
                _vec_spec(D_H), _vec_spec(D_H), _full_spec(D_H, D_H),
                _row_spec(128), _row_spec(128), _row_spec(128), _row_spec(128)],
      out_specs=[_row_spec(128)] * 4,
      out_shape=[jax.ShapeDtypeStruct((N, 128), jnp.float32)] * 4,
  )(dinv, W0, b0, g0, be0, W1, *aggs, *us)


def _mid_body(dinv_ref, b_ref, g_ref, be_ref, W_ref, *rest):
  aggs, us, outs = rest[:4], rest[4:8], rest[8:]
  dinv = dinv_ref[...]
  agg = jnp.concatenate([r[...] for r in aggs], axis=1)
  u = jnp.concatenate([r[...] for r in us], axis=1)
  conv = (agg + u) * dinv + b_ref[...]
  h = _ln_elu(conv, g_ref[...], be_ref[...])
  un = jnp.dot(h, W_ref[...], preferred_element_type=jnp.float32) * dinv
  for k, o in enumerate(outs):
    o[...] = un[:, k * 128:(k + 1) * 128]


def _mid_layer(dinv, b, g, be, W, aggs, us, nout):
  return pl.pallas_call(
      _mid_body,
      grid=(GRID,),
      in_specs=[_row_spec(1), _vec_spec(D_H), _vec_spec(D_H), _vec_spec(D_H),
                _full_spec(D_H, W.shape[1])] + [_row_spec(128)] * 8,
      out_specs=[_row_spec(128)] * nout,
      out_shape=[jax.ShapeDtypeStruct((N, 128), jnp.float32)] * nout,
  )(dinv, b, g, be, W, *aggs, *us)


def _final_body(dinv_ref, b3_ref, x_ref, a0, a1, u0, u1, out_ref):
  agg = jnp.concatenate([a0[...], a1[...]], axis=1)
  u = jnp.concatenate([u0[...], u1[...]], axis=1)
  out_ref[...] = (agg + u) * dinv_ref[...] + b3_ref[...] + x_ref[...]


def _final(dinv, b3, x, aggs, us):
  return pl.pallas_call(
      _final_body,
      grid=(GRID,),
      in_specs=[_row_spec(1), _vec_spec(D_IN), _row_spec(D_IN)]
               + [_row_spec(128)] * 4,
      out_specs=_row_spec(D_IN),
      out_shape=jax.ShapeDtypeStruct((N, D_IN), jnp.float32),
  )(dinv, b3, x, *aggs, *us)


def kernel(x, edge_index, W0, b0, W1, b1, W2, b2, W3, b3,
           g0, be0, g1, be1, g2, be2):
  src = edge_index[0].reshape(NS, EPT)
  dst = edge_index[1].reshape(NS, EPT)
  npad = EPTP - EPT
  pad_src = jnp.broadcast_to((jnp.arange(npad, dtype=jnp.int32) * 521) % N,
                             (NS, npad))
  pad_dst = jnp.broadcast_to(N + (jnp.arange(npad, dtype=jnp.int32) % NS),
                             (NS, npad))
  srcp = jnp.concatenate([src, pad_src], axis=1).reshape(NS, NBLK, EB)
  dstp = jnp.concatenate([dst, pad_dst], axis=1).reshape(NS, NBLK, EB)

  ones_eb = jnp.ones((EB,), jnp.float32)
  zeros_deg = jnp.zeros((DEG_PT,), jnp.float32)
  zeros_acc = jnp.zeros((ZROWS, 128), jnp.float32)

  degp = _deg_kernel(dstp, ones_eb, zeros_deg).reshape(NC, DEG_P)
  dinv_row = _finalize_deg(degp)
  dinv = dinv_row.reshape(DEG_P, 1)[:N]

  ux = _prep(dinv, x)
  aggx = _agg2(srcp, dstp, zeros_acc, *ux)
  u1 = _layer0(dinv, W0, b0, g0, be0, W1, aggx, ux)
  agg1 = _agg4(srcp, dstp, zeros_acc, *u1)
  u2 = _mid_layer(dinv, b1, g1, be1, W2, agg1, u1, 4)
  agg2 = _agg4(srcp, dstp, zeros_acc, *u2)
  u3 = _mid_layer(dinv, b2, g2, be2, W3, agg2, u2, 2)
  agg3 = _agg2(srcp, dstp, zeros_acc, *u3)
  return _final(dinv, b3, x, agg3, u3)


# speedup vs baseline: 1.1746x; 1.1324x over previous
import functools

import jax
import jax.numpy as jnp
from jax import lax
from jax.experimental import pallas as pl
from jax.experimental.pallas import tpu as pltpu
from jax.experimental.pallas import tpu_sc as plsc

N = 10000
E = 160000
D_IN = 256
D_H = 512

NC = 2
NS = 16
EB = 64

EPT = E // NS
CHB = 32
NCHK = 5
NBLK = CHB * NCHK
EPTP = NBLK * EB
ZROWS = 632
ACC_ROWS = NS * ZROWS
DEG_PT = 640
DEG_P = NS * DEG_PT

_mesh = lambda: plsc.VectorSubcoreMesh(core_axis_name="c", subcore_axis_name="s")


def _deg_body(dstp, ones_hbm, zeros_hbm, out, didx_all, ones_v, acc, sem):
  c = lax.axis_index("c")
  s = lax.axis_index("s")
  pltpu.sync_copy(ones_hbm, ones_v)
  pltpu.sync_copy(zeros_hbm, acc.at[pl.ds(s * DEG_PT, DEG_PT)])
  pltpu.sync_copy(dstp.at[s], didx_all)
  plsc.subcore_barrier()

  def step(b, carry):
    pltpu.async_copy(ones_v, acc.at[didx_all.at[b]], sem, add=True).wait()
    return carry

  half = NBLK // 2

  @pl.when(c == 0)
  def _():
    lax.fori_loop(0, half, step, 0)

  @pl.when(c == 1)
  def _():
    lax.fori_loop(half, NBLK, step, 0)

  plsc.subcore_barrier()
  pltpu.sync_copy(acc.at[pl.ds(s * DEG_PT, DEG_PT)],
                  out.at[c, 0, pl.ds(s * DEG_PT, DEG_PT)])


_deg_kernel = functools.partial(
    pl.kernel,
    out_type=jax.ShapeDtypeStruct((NC, 1, DEG_P), jnp.float32),
    mesh=_mesh(),
    scratch_types=[
        pltpu.VMEM((NBLK, EB), jnp.int32),
        pltpu.VMEM((EB,), jnp.float32),
        pltpu.VMEM_SHARED((DEG_P,), jnp.float32),
        pltpu.SemaphoreType.DMA,
    ],
)(_deg_body)


def _make_agg(nch):
  per_core = nch // NC

  def body(srcp, dstp, zeros_hbm, *rest):
    chunks = rest[:nch]
    outs = rest[nch:2 * nch]
    (sidx0, didx0, sidx1, didx1, buf0, buf1, buf2, buf3,
     acc, gsem, ssem, isem) = rest[2 * nch:]
    bufs = (buf0, buf1, buf2, buf3)
    idxsets = ((sidx0, didx0), (sidx1, didx1))
    c = lax.axis_index("c")
    s = lax.axis_index("s")

    def stage_descs(ci, iset):
      return (pltpu.make_async_copy(srcp.at[s, pl.ds(ci * CHB, CHB)],
                                    iset[0], isem),
              pltpu.make_async_copy(dstp.at[s, pl.ds(ci * CHB, CHB)],
                                    iset[1], isem))

    def chunk_pass(table, out):
      pltpu.sync_copy(zeros_hbm, acc.at[pl.ds(s * ZROWS, ZROWS), :])
      for d in stage_descs(0, idxsets[0]):
        d.start()
      plsc.subcore_barrier()

      def g_desc(iset, j, buf):
        return pltpu.make_async_copy(table.at[iset[0].at[j]], buf, gsem)

      def s_desc(iset, j, buf):
        return pltpu.make_async_copy(buf, acc.at[iset[1].at[j]], ssem)

      for ci in range(NCHK):
        iset = idxsets[ci % 2]
        for d in stage_descs(ci, iset):
          d.wait()
        if ci + 1 < NCHK:
          for d in stage_descs(ci + 1, idxsets[(ci + 1) % 2]):
            d.start()
        g_desc(iset, 0, bufs[0]).start()
        g_desc(iset, 1, bufs[1]).start()

        def quadstep(p, carry, iset=iset):
          for q in range(4):
            j = p * 4 + q

            @pl.when(j >= 2)
            def _():
              s_desc(iset, j - 2, bufs[(q - 2) % 4]).wait()

            g_desc(iset, j, bufs[q]).wait()

            @pl.when(j + 2 < CHB)
            def _():
              g_desc(iset, j + 2, bufs[(q + 2) % 4]).start()

            s_desc(iset, j, bufs[q]).start(add=True)
          return carry

        lax.fori_loop(0, CHB // 4, quadstep, 0)
        s_desc(iset, CHB - 2, bufs[(CHB - 2) % 4]).wait()
        s_desc(iset, CHB - 1, bufs[(CHB - 1) % 4]).wait()

      plsc.subcore_barrier()
      pltpu.sync_copy(acc.at[pl.ds(s * ZROWS, ZROWS), :],
                      out.at[pl.ds(s * ZROWS, ZROWS), :])
      plsc.subcore_barrier()

    for cc in range(NC):
      @pl.when(c == cc)
      def _(cc=cc):
        for j in range(per_core):
          k = cc * per_core + j
          chunk_pass(chunks[k], outs[k])

  return pl.kernel(
      body,
      out_type=[jax.ShapeDtypeStruct((ACC_ROWS, 128), jnp.float32)
                for _ in range(nch)],
      mesh=_mesh(),
      scratch_types=[
          pltpu.VMEM((CHB, EB), jnp.int32),
          pltpu.VMEM((CHB, EB), jnp.int32),
          pltpu.VMEM((CHB, EB), jnp.int32),
          pltpu.VMEM((CHB, EB), jnp.int32),
          pltpu.VMEM((EB, 128), jnp.float32),
          pltpu.VMEM((EB, 128), jnp.float32),
          pltpu.VMEM((EB, 128), jnp.float32),
          pltpu.VMEM((EB, 128), jnp.float32),
          pltpu.VMEM_SHARED((ACC_ROWS, 128), jnp.float32),
          pltpu.SemaphoreType.DMA,
          pltpu.SemaphoreType.DMA,
          pltpu.SemaphoreType.DMA,
      ],
  )


_agg2 = _make_agg(2)
_agg4 = _make_agg(4)


RB = 1000
GRID = N // RB

def _row_spec(w):
  return pl.BlockSpec((RB, w), lambda i: (i, 0))

def _full_spec(h, w):
  return pl.BlockSpec((h, w), lambda i: (0, 0))

def _vec_spec(n):
  return pl.BlockSpec((n,), lambda i: (0,))


def _finalize_deg_body(degp_ref, out_ref):
  deg = degp_ref[0:1, :] + degp_ref[1:2, :] + 1.0
  out_ref[...] = lax.rsqrt(deg)


def _finalize_deg(degp):
  return pl.pallas_call(
      _finalize_deg_body,
      out_shape=jax.ShapeDtypeStruct((1, DEG_P), jnp.float32),
  )(degp)


def _prep_body(dinv_ref, x_ref, o0, o1):
  ux = x_ref[...] * dinv_ref[...]
  o0[...] = ux[:, :128]
  o1[...] = ux[:, 128:]


def _prep(dinv, x):
  return pl.pallas_call(
      _prep_body,
      grid=(GRID,),
      in_specs=[_row_spec(1), _row_spec(D_IN)],
      out_specs=[_row_spec(128), _row_spec(128)],
      out_shape=[jax.ShapeDtypeStruct((N, 128), jnp.float32)] * 2,
  )(dinv, x)


def _ln_elu(v, g, b):
  mu = jnp.mean(v, axis=1, keepdims=True)
  d = v - mu
  var = jnp.mean(d * d, axis=1, keepdims=True)
  hn = d * lax.rsqrt(var + 1e-5) * g + b
  return jnp.where(hn > 0, hn, jnp.exp(hn) - 1.0)


def _layer0_body(dinv_ref, W0_ref, b0_ref, g0_ref, be0_ref, W1_ref,
                 a0, a1, u0, u1, *outs):
  dinv = dinv_ref[...]
  agg = jnp.concatenate([a0[...], a1[...]], axis=1)
  u = jnp.concatenate([u0[...], u1[...]], axis=1)
  sx = (agg + u) * dinv
  pre = jnp.dot(sx, W0_ref[...], preferred_element_type=jnp.float32) + b0_ref[...]
  h = _ln_elu(pre, g0_ref[...], be0_ref[...])
  un = jnp.dot(h, W1_ref[...], preferred_element_type=jnp.float32) * dinv
  for k, o in enumerate(outs):
    o[...] = un[:, k * 128:(k + 1) * 128]


def _layer0(dinv, W0, b0, g0, be0, W1, aggs, us):
  return pl.pallas_call(
      _layer0_body,
      grid=(GRID,),
      in_specs=[_row_spec(1), _full_spec(D_IN, D_H), _vec_spec(D_H),
                _vec_spec(D_H), _vec_spec(D_H), _full_spec(D_H, D_H),
                _row_spec(128), _row_spec(128), _row_spec(128), _row_spec(128)],
      out_specs=[_row_spec(128)] * 4,
      out_shape=[jax.ShapeDtypeStruct((N, 128), jnp.float32)] * 4,
  )(dinv, W0, b0, g0, be0, W1, *aggs, *us)


def _mid_body(dinv_ref, b_ref, g_ref, be_ref, W_ref, *rest):
  aggs, us, outs = rest[:4], rest[4:8], rest[8:]
  dinv = dinv_ref[...]
  agg = jnp.concatenate([r[...] for r in aggs], axis=1)
  u = jnp.concatenate([r[...] for r in us], axis=1)
  conv = (agg + u) * dinv + b_ref[...]
  h = _ln_elu(conv, g_ref[...], be_ref[...])
  un = jnp.dot(h, W_ref[...], preferred_element_type=jnp.float32) * dinv
  for k, o in enumerate(outs):
    o[...] = un[:, k * 128:(k + 1) * 128]


def _mid_layer(dinv, b, g, be, W, aggs, us, nout):
  return pl.pallas_call(
      _mid_body,
      grid=(GRID,),
      in_specs=[_row_spec(1), _vec_spec(D_H), _vec_spec(D_H), _vec_spec(D_H),
                _full_spec(D_H, W.shape[1])] + [_row_spec(128)] * 8,
      out_specs=[_row_spec(128)] * nout,
      out_shape=[jax.ShapeDtypeStruct((N, 128), jnp.float32)] * nout,
  )(dinv, b, g, be, W, *aggs, *us)


def _final_body(dinv_ref, b3_ref, x_ref, a0, a1, u0, u1, out_ref):
  agg = jnp.concatenate([a0[...], a1[...]], axis=1)
  u = jnp.concatenate([u0[...], u1[...]], axis=1)
  out_ref[...] = (agg + u) * dinv_ref[...] + b3_ref[...] + x_ref[...]


def _final(dinv, b3, x, aggs, us):
  return pl.pallas_call(
      _final_body,
      grid=(GRID,),
      in_specs=[_row_spec(1), _vec_spec(D_IN), _row_spec(D_IN)]
               + [_row_spec(128)] * 4,
      out_specs=_row_spec(D_IN),
      out_shape=jax.ShapeDtypeStruct((N, D_IN), jnp.float32),
  )(dinv, b3, x, *aggs, *us)


def kernel(x, edge_index, W0, b0, W1, b1, W2, b2, W3, b3,
           g0, be0, g1, be1, g2, be2):
  src = edge_index[0].reshape(NS, EPT)
  dst = edge_index[1].reshape(NS, EPT)
  npad = EPTP - EPT
  pad_src = jnp.broadcast_to((jnp.arange(npad, dtype=jnp.int32) * 521) % N,
                             (NS, npad))
  pad_dst = jnp.broadcast_to(N + (jnp.arange(npad, dtype=jnp.int32) % NS),
                             (NS, npad))
  srcp = jnp.concatenate([src, pad_src], axis=1).reshape(NS, NBLK, EB)
  dstp = jnp.concatenate([dst, pad_dst], axis=1).reshape(NS, NBLK, EB)

  ones_eb = jnp.ones((EB,), jnp.float32)
  zeros_deg = jnp.zeros((DEG_PT,), jnp.float32)
  zeros_acc = jnp.zeros((ZROWS, 128), jnp.float32)

  degp = _deg_kernel(dstp, ones_eb, zeros_deg).reshape(NC, DEG_P)
  dinv_row = _finalize_deg(degp)
  dinv = dinv_row.reshape(DEG_P, 1)[:N]

  ux = _prep(dinv, x)
  aggx = _agg2(srcp, dstp, zeros_acc, *ux)
  u1 = _layer0(dinv, W0, b0, g0, be0, W1, aggx, ux)
  agg1 = _agg4(srcp, dstp, zeros_acc, *u1)
  u2 = _mid_layer(dinv, b1, g1, be1, W2, agg1, u1, 4)
  agg2 = _agg4(srcp, dstp, zeros_acc, *u2)
  u3 = _mid_layer(dinv, b2, g2, be2, W3, agg2, u2, 2)
  agg3 = _agg2(srcp, dstp, zeros_acc, *u3)
  return _final(dinv, b3, x, agg3, u3)


# speedup vs baseline: 1.2087x; 1.0290x over previous
import functools

import jax
import jax.numpy as jnp
from jax import lax
from jax.experimental import pallas as pl
from jax.experimental.pallas import tpu as pltpu
from jax.experimental.pallas import tpu_sc as plsc

N = 10000
E = 160000
D_IN = 256
D_H = 512

NC = 2
NS = 16
EB = 32

EPT = E // NS
NBUF = 8
CHB = 64
NCHK = 5
NBLK = CHB * NCHK
EPTP = NBLK * EB
ZROWS = 632
ACC_ROWS = NS * ZROWS
DEG_PT = 640
DEG_P = NS * DEG_PT

_mesh = lambda: plsc.VectorSubcoreMesh(core_axis_name="c", subcore_axis_name="s")


def _deg_body(dstp, ones_hbm, zeros_hbm, out, didx_all, ones_v, acc, sem):
  c = lax.axis_index("c")
  s = lax.axis_index("s")
  pltpu.sync_copy(ones_hbm, ones_v)
  pltpu.sync_copy(zeros_hbm, acc.at[pl.ds(s * DEG_PT, DEG_PT)])
  pltpu.sync_copy(dstp.at[s], didx_all)
  plsc.subcore_barrier()

  def step(b, carry):
    pltpu.async_copy(ones_v, acc.at[didx_all.at[b]], sem, add=True).wait()
    return carry

  half = NBLK // 2

  @pl.when(c == 0)
  def _():
    lax.fori_loop(0, half, step, 0)

  @pl.when(c == 1)
  def _():
    lax.fori_loop(half, NBLK, step, 0)

  plsc.subcore_barrier()
  pltpu.sync_copy(acc.at[pl.ds(s * DEG_PT, DEG_PT)],
                  out.at[c, 0, pl.ds(s * DEG_PT, DEG_PT)])


_deg_kernel = functools.partial(
    pl.kernel,
    out_type=jax.ShapeDtypeStruct((NC, 1, DEG_P), jnp.float32),
    mesh=_mesh(),
    scratch_types=[
        pltpu.VMEM((NBLK, EB), jnp.int32),
        pltpu.VMEM((EB,), jnp.float32),
        pltpu.VMEM_SHARED((DEG_P,), jnp.float32),
        pltpu.SemaphoreType.DMA,
    ],
)(_deg_body)


def _make_agg(nch):
  per_core = nch // NC

  def body(srcp, dstp, zeros_hbm, *rest):
    chunks = rest[:nch]
    outs = rest[nch:2 * nch]
    sidx0, didx0 = rest[2 * nch:2 * nch + 2]
    bufs = rest[2 * nch + 2:2 * nch + 2 + NBUF]
    acc, gsem, ssem, isem = rest[2 * nch + 2 + NBUF:]
    idxsets = ((sidx0, didx0), (sidx0, didx0))
    c = lax.axis_index("c")
    s = lax.axis_index("s")

    def stage_descs(ci, iset):
      return (pltpu.make_async_copy(srcp.at[s, pl.ds(ci * CHB, CHB)],
                                    iset[0], isem),
              pltpu.make_async_copy(dstp.at[s, pl.ds(ci * CHB, CHB)],
                                    iset[1], isem))

    def chunk_pass(table, out):
      pltpu.sync_copy(zeros_hbm, acc.at[pl.ds(s * ZROWS, ZROWS), :])
      for d in stage_descs(0, idxsets[0]):
        d.start()
      plsc.subcore_barrier()

      def g_desc(iset, j, buf):
        return pltpu.make_async_copy(table.at[iset[0].at[j]], buf, gsem)

      def s_desc(iset, j, buf):
        return pltpu.make_async_copy(buf, acc.at[iset[1].at[j]], ssem)

      for ci in range(NCHK):
        iset = idxsets[ci % 2]
        for d in stage_descs(ci, iset):
          d.wait()
        half = NBUF // 2
        for q in range(half):
          g_desc(iset, q, bufs[q]).start()

        def rotstep(p, carry, iset=iset):
          for q in range(NBUF):
            j = p * NBUF + q

            @pl.when(j >= half)
            def _():
              s_desc(iset, j - half, bufs[(q - half) % NBUF]).wait()

            g_desc(iset, j, bufs[q]).wait()

            @pl.when(j + half < CHB)
            def _():
              g_desc(iset, j + half, bufs[(q + half) % NBUF]).start()

            s_desc(iset, j, bufs[q]).start(add=True)
          return carry

        lax.fori_loop(0, CHB // NBUF, rotstep, 0)
        for t in range(half):
          b = CHB - half + t
          s_desc(iset, b, bufs[b % NBUF]).wait()
        if ci + 1 < NCHK:
          for d in stage_descs(ci + 1, idxsets[(ci + 1) % 2]):
            d.start()

      plsc.subcore_barrier()
      pltpu.sync_copy(acc.at[pl.ds(s * ZROWS, ZROWS), :],
                      out.at[pl.ds(s * ZROWS, ZROWS), :])
      plsc.subcore_barrier()

    for cc in range(NC):
      @pl.when(c == cc)
      def _(cc=cc):
        for j in range(per_core):
          k = cc * per_core + j
          chunk_pass(chunks[k], outs[k])

  return pl.kernel(
      body,
      out_type=[jax.ShapeDtypeStruct((ACC_ROWS, 128), jnp.float32)
                for _ in range(nch)],
      mesh=_mesh(),
      scratch_types=[
          pltpu.VMEM((CHB, EB), jnp.int32),
          pltpu.VMEM((CHB, EB), jnp.int32),
      ] + [pltpu.VMEM((EB, 128), jnp.float32) for _ in range(NBUF)] + [
          pltpu.VMEM_SHARED((ACC_ROWS, 128), jnp.float32),
          pltpu.SemaphoreType.DMA,
          pltpu.SemaphoreType.DMA,
          pltpu.SemaphoreType.DMA,
      ],
  )


_agg2 = _make_agg(2)
_agg4 = _make_agg(4)


RB = 1000
GRID = N // RB

def _row_spec(w):
  return pl.BlockSpec((RB, w), lambda i: (i, 0))

def _full_spec(h, w):
  return pl.BlockSpec((h, w), lambda i: (0, 0))

def _vec_spec(n):
  return pl.BlockSpec((n,), lambda i: (0,))


def _finalize_deg_body(degp_ref, out_ref):
  deg = degp_ref[0:1, :] + degp_ref[1:2, :] + 1.0
  out_ref[...] = lax.rsqrt(deg)


def _finalize_deg(degp):
  return pl.pallas_call(
      _finalize_deg_body,
      out_shape=jax.ShapeDtypeStruct((1, DEG_P), jnp.float32),
  )(degp)


def _prep_body(dinv_ref, x_ref, o0, o1):
  ux = x_ref[...] * dinv_ref[...]
  o0[...] = ux[:, :128]
  o1[...] = ux[:, 128:]


def _prep(dinv, x):
  return pl.pallas_call(
      _prep_body,
      grid=(GRID,),
      in_specs=[_row_spec(1), _row_spec(D_IN)],
      out_specs=[_row_spec(128), _row_spec(128)],
      out_shape=[jax.ShapeDtypeStruct((N, 128), jnp.float32)] * 2,
  )(dinv, x)


def _ln_elu(v, g, b):
  mu = jnp.mean(v, axis=1, keepdims=True)
  d = v - mu
  var = jnp.mean(d * d, axis=1, keepdims=True)
  hn = d * lax.rsqrt(var + 1e-5) * g + b
  return jnp.where(hn > 0, hn, jnp.exp(hn) - 1.0)


def _layer0_body(dinv_ref, W0_ref, b0_ref, g0_ref, be0_ref, W1_ref,
                 a0, a1, u0, u1, *outs):
  dinv = dinv_ref[...]
  agg = jnp.concatenate([a0[...], a1[...]], axis=1)
  u = jnp.concatenate([u0[...], u1[...]], axis=1)
  sx = (agg + u) * dinv
  pre = jnp.dot(sx, W0_ref[...], preferred_element_type=jnp.float32) + b0_ref[...]
  h = _ln_elu(pre, g0_ref[...], be0_ref[...])
  un = jnp.dot(h, W1_ref[...], preferred_element_type=jnp.float32) * dinv
  for k, o in enumerate(outs):
    o[...] = un[:, k * 128:(k + 1) * 128]


def _layer0(dinv, W0, b0, g0, be0, W1, aggs, us):
  return pl.pallas_call(
      _layer0_body,
      grid=(GRID,),
      in_specs=[_row_spec(1), _full_spec(D_IN, D_H), _vec_spec(D_H),
                _vec_spec(D_H), _vec_spec(D_H), _full_spec(D_H, D_H),
                _row_spec(128), _row_spec(128), _row_spec(128), _row_spec(128)],
      out_specs=[_row_spec(128)] * 4,
      out_shape=[jax.ShapeDtypeStruct((N, 128), jnp.float32)] * 4,
  )(dinv, W0, b0, g0, be0, W1, *aggs, *us)


def _mid_body(dinv_ref, b_ref, g_ref, be_ref, W_ref, *rest):
  aggs, us, outs = rest[:4], rest[4:8], rest[8:]
  dinv = dinv_ref[...]
  agg = jnp.concatenate([r[...] for r in aggs], axis=1)
  u = jnp.concatenate([r[...] for r in us], axis=1)
  conv = (agg + u) * dinv + b_ref[...]
  h = _ln_elu(conv, g_ref[...], be_ref[...])
  un = jnp.dot(h, W_ref[...], preferred_element_type=jnp.float32) * dinv
  for k, o in enumerate(outs):
    o[...] = un[:, k * 128:(k + 1) * 128]


def _mid_layer(dinv, b, g, be, W, aggs, us, nout):
  return pl.pallas_call(
      _mid_body,
      grid=(GRID,),
      in_specs=[_row_spec(1), _vec_spec(D_H), _vec_spec(D_H), _vec_spec(D_H),
                _full_spec(D_H, W.shape[1])] + [_row_spec(128)] * 8,
      out_specs=[_row_spec(128)] * nout,
      out_shape=[jax.ShapeDtypeStruct((N, 128), jnp.float32)] * nout,
  )(dinv, b, g, be, W, *aggs, *us)


def _final_body(dinv_ref, b3_ref, x_ref, a0, a1, u0, u1, out_ref):
  agg = jnp.concatenate([a0[...], a1[...]], axis=1)
  u = jnp.concatenate([u0[...], u1[...]], axis=1)
  out_ref[...] = (agg + u) * dinv_ref[...] + b3_ref[...] + x_ref[...]


def _final(dinv, b3, x, aggs, us):
  return pl.pallas_call(
      _final_body,
      grid=(GRID,),
      in_specs=[_row_spec(1), _vec_spec(D_IN), _row_spec(D_IN)]
               + [_row_spec(128)] * 4,
      out_specs=_row_spec(D_IN),
      out_shape=jax.ShapeDtypeStruct((N, D_IN), jnp.float32),
  )(dinv, b3, x, *aggs, *us)


def kernel(x, edge_index, W0, b0, W1, b1, W2, b2, W3, b3,
           g0, be0, g1, be1, g2, be2):
  src = edge_index[0].reshape(NS, EPT)
  dst = edge_index[1].reshape(NS, EPT)
  npad = EPTP - EPT
  pad_src = jnp.broadcast_to((jnp.arange(npad, dtype=jnp.int32) * 521) % N,
                             (NS, npad))
  pad_dst = jnp.broadcast_to(N + (jnp.arange(npad, dtype=jnp.int32) % NS),
                             (NS, npad))
  srcp = jnp.concatenate([src, pad_src], axis=1).reshape(NS, NBLK, EB)
  dstp = jnp.concatenate([dst, pad_dst], axis=1).reshape(NS, NBLK, EB)

  ones_eb = jnp.ones((EB,), jnp.float32)
  zeros_deg = jnp.zeros((DEG_PT,), jnp.float32)
  zeros_acc = jnp.zeros((ZROWS, 128), jnp.float32)

  degp = _deg_kernel(dstp, ones_eb, zeros_deg).reshape(NC, DEG_P)
  dinv_row = _finalize_deg(degp)
  dinv = dinv_row.reshape(DEG_P, 1)[:N]

  ux = _prep(dinv, x)
  aggx = _agg2(srcp, dstp, zeros_acc, *ux)
  u1 = _layer0(dinv, W0, b0, g0, be0, W1, aggx, ux)
  agg1 = _agg4(srcp, dstp, zeros_acc, *u1)
  u2 = _mid_layer(dinv, b1, g1, be1, W2, agg1, u1, 4)
  agg2 = _agg4(srcp, dstp, zeros_acc, *u2)
  u3 = _mid_layer(dinv, b2, g2, be2, W3, agg2, u2, 2)
  agg3 = _agg2(srcp, dstp, zeros_acc, *u3)
  return _final(dinv, b3, x, agg3, u3)


# speedup vs baseline: 1.2125x; 1.0031x over previous
import functools

import jax
import jax.numpy as jnp
from jax import lax
from jax.experimental import pallas as pl
from jax.experimental.pallas import tpu as pltpu
from jax.experimental.pallas import tpu_sc as plsc

N = 10000
E = 160000
D_IN = 256
D_H = 512

NC = 2
NS = 16
EB = 32

EPT = E // NS
NBUF = 8
CHB = 64
NCHK = 5
NBLK = CHB * NCHK
EPTP = NBLK * EB
ZROWS = 632
ACC_ROWS = NS * ZROWS
DEG_PT = 640
DEG_P = NS * DEG_PT

_mesh = lambda: plsc.VectorSubcoreMesh(core_axis_name="c", subcore_axis_name="s")


def _deg_body(dstp, ones_hbm, zeros_hbm, out, didx_all, ones_v, acc, sem):
  c = lax.axis_index("c")
  s = lax.axis_index("s")
  pltpu.sync_copy(ones_hbm, ones_v)
  pltpu.sync_copy(zeros_hbm, acc.at[pl.ds(s * DEG_PT, DEG_PT)])
  pltpu.sync_copy(dstp.at[s], didx_all)
  plsc.subcore_barrier()

  def step(b, carry):
    pltpu.async_copy(ones_v, acc.at[didx_all.at[b]], sem, add=True).wait()
    return carry

  half = NBLK // 2

  @pl.when(c == 0)
  def _():
    lax.fori_loop(0, half, step, 0)

  @pl.when(c == 1)
  def _():
    lax.fori_loop(half, NBLK, step, 0)

  plsc.subcore_barrier()
  pltpu.sync_copy(acc.at[pl.ds(s * DEG_PT, DEG_PT)],
                  out.at[c, 0, pl.ds(s * DEG_PT, DEG_PT)])


_deg_kernel = functools.partial(
    pl.kernel,
    out_type=jax.ShapeDtypeStruct((NC, 1, DEG_P), jnp.float32),
    mesh=_mesh(),
    scratch_types=[
        pltpu.VMEM((NBLK, EB), jnp.int32),
        pltpu.VMEM((EB,), jnp.float32),
        pltpu.VMEM_SHARED((DEG_P,), jnp.float32),
        pltpu.SemaphoreType.DMA,
    ],
)(_deg_body)


def _make_agg(nch):
  per_core = nch // NC

  def body(srcp, dstp, zeros_hbm, *rest):
    chunks = rest[:nch]
    outs = rest[nch:2 * nch]
    sidx0, didx0 = rest[2 * nch:2 * nch + 2]
    bufs = rest[2 * nch + 2:2 * nch + 2 + NBUF]
    acc, gsem, ssem, isem, zsem = rest[2 * nch + 2 + NBUF:]
    idxsets = ((sidx0, didx0), (sidx0, didx0))
    c = lax.axis_index("c")
    s = lax.axis_index("s")

    def stage_descs(ci, iset):
      return (pltpu.make_async_copy(srcp.at[s, pl.ds(ci * CHB, CHB)],
                                    iset[0], isem),
              pltpu.make_async_copy(dstp.at[s, pl.ds(ci * CHB, CHB)],
                                    iset[1], isem))

    def chunk_pass(table, out):
      zdesc = pltpu.make_async_copy(
          zeros_hbm, acc.at[pl.ds(s * ZROWS, ZROWS), :], zsem)
      zdesc.start()
      for d in stage_descs(0, idxsets[0]):
        d.start()
      zdesc.wait()
      plsc.subcore_barrier()

      def g_desc(iset, j, buf):
        return pltpu.make_async_copy(table.at[iset[0].at[j]], buf, gsem)

      def s_desc(iset, j, buf):
        return pltpu.make_async_copy(buf, acc.at[iset[1].at[j]], ssem)

      for ci in range(NCHK):
        iset = idxsets[ci % 2]
        for d in stage_descs(ci, iset):
          d.wait()
        half = NBUF // 2
        for q in range(half):
          g_desc(iset, q, bufs[q]).start()

        def rotstep(p, carry, iset=iset):
          for q in range(NBUF):
            j = p * NBUF + q

            @pl.when(j >= half)
            def _():
              s_desc(iset, j - half, bufs[(q - half) % NBUF]).wait()

            g_desc(iset, j, bufs[q]).wait()

            @pl.when(j + half < CHB)
            def _():
              g_desc(iset, j + half, bufs[(q + half) % NBUF]).start()

            s_desc(iset, j, bufs[q]).start(add=True)
          return carry

        lax.fori_loop(0, CHB // NBUF, rotstep, 0)
        for t in range(half):
          b = CHB - half + t
          s_desc(iset, b, bufs[b % NBUF]).wait()
        if ci + 1 < NCHK:
          for d in stage_descs(ci + 1, idxsets[(ci + 1) % 2]):
            d.start()

      plsc.subcore_barrier()
      pltpu.sync_copy(acc.at[pl.ds(s * ZROWS, ZROWS), :],
                      out.at[pl.ds(s * ZROWS, ZROWS), :])
      plsc.subcore_barrier()

    for cc in range(NC):
      @pl.when(c == cc)
      def _(cc=cc):
        for j in range(per_core):
          k = cc * per_core + j
          chunk_pass(chunks[k], outs[k])

  return pl.kernel(
      body,
      out_type=[jax.ShapeDtypeStruct((ACC_ROWS, 128), jnp.float32)
                for _ in range(nch)],
      mesh=_mesh(),
      scratch_types=[
          pltpu.VMEM((CHB, EB), jnp.int32),
          pltpu.VMEM((CHB, EB), jnp.int32),
      ] + [pltpu.VMEM((EB, 128), jnp.float32) for _ in range(NBUF)] + [
          pltpu.VMEM_SHARED((ACC_ROWS, 128), jnp.float32),
          pltpu.SemaphoreType.DMA,
          pltpu.SemaphoreType.DMA,
          pltpu.SemaphoreType.DMA,
          pltpu.SemaphoreType.DMA,
      ],
  )


_agg2 = _make_agg(2)
_agg4 = _make_agg(4)


RB = 1000
GRID = N // RB

def _row_spec(w):
  return pl.BlockSpec((RB, w), lambda i: (i, 0))

def _full_spec(h, w):
  return pl.BlockSpec((h, w), lambda i: (0, 0))

def _vec_spec(n):
  return pl.BlockSpec((n,), lambda i: (0,))


def _finalize_deg_body(degp_ref, out_ref):
  deg = degp_ref[0:1, :] + degp_ref[1:2, :] + 1.0
  out_ref[...] = lax.rsqrt(deg)


def _finalize_deg(degp):
  return pl.pallas_call(
      _finalize_deg_body,
      out_shape=jax.ShapeDtypeStruct((1, DEG_P), jnp.float32),
  )(degp)


def _prep_body(dinv_ref, x_ref, o0, o1):
  ux = x_ref[...] * dinv_ref[...]
  o0[...] = ux[:, :128]
  o1[...] = ux[:, 128:]


def _prep(dinv, x):
  return pl.pallas_call(
      _prep_body,
      grid=(GRID,),
      in_specs=[_row_spec(1), _row_spec(D_IN)],
      out_specs=[_row_spec(128), _row_spec(128)],
      out_shape=[jax.ShapeDtypeStruct((N, 128), jnp.float32)] * 2,
  )(dinv, x)


def _ln_elu(v, g, b):
  mu = jnp.mean(v, axis=1, keepdims=True)
  d = v - mu
  var = jnp.mean(d * d, axis=1, keepdims=True)
  hn = d * lax.rsqrt(var + 1e-5) * g + b
  return jnp.where(hn > 0, hn, jnp.exp(hn) - 1.0)


def _layer0_body(dinv_ref, W0_ref, b0_ref, g0_ref, be0_ref, W1_ref,
                 a0, a1, u0, u1, *outs):
  dinv = dinv_ref[...]
  agg = jnp.concatenate([a0[...], a1[...]], axis=1)
  u = jnp.concatenate([u0[...], u1[...]], axis=1)
  sx = (agg + u) * dinv
  pre = jnp.dot(sx, W0_ref[...], preferred_element_type=jnp.float32) + b0_ref[...]
  h = _ln_elu(pre, g0_ref[...], be0_ref[...])
  un = jnp.dot(h, W1_ref[...], preferred_element_type=jnp.float32) * dinv
  for k, o in enumerate(outs):
    o[...] = un[:, k * 128:(k + 1) * 128]


def _layer0(dinv, W0, b0, g0, be0, W1, aggs, us):
  return pl.pallas_call(
      _layer0_body,
      grid=(GRID,),
      in_specs=[_row_spec(1), _full_spec(D_IN, D_H), _vec_spec(D_H),
                _vec_spec(D_H), _vec_spec(D_H), _full_spec(D_H, D_H),
                _row_spec(128), _row_spec(128), _row_spec(128), _row_spec(128)],
      out_specs=[_row_spec(128)] * 4,
      out_shape=[jax.ShapeDtypeStruct((N, 128), jnp.float32)] * 4,
  )(dinv, W0, b0, g0, be0, W1, *aggs, *us)


def _mid_body(dinv_ref, b_ref, g_ref, be_ref, W_ref, *rest):
  aggs, us, outs = rest[:4], rest[4:8], rest[8:]
  dinv = dinv_ref[...]
  agg = jnp.concatenate([r[...] for r in aggs], axis=1)
  u = jnp.concatenate([r[...] for r in us], axis=1)
  conv = (agg + u) * dinv + b_ref[...]
  h = _ln_elu(conv, g_ref[...], be_ref[...])
  un = jnp.dot(h, W_ref[...], preferred_element_type=jnp.float32) * dinv
  for k, o in enumerate(outs):
    o[...] = un[:, k * 128:(k + 1) * 128]


def _mid_layer(dinv, b, g, be, W, aggs, us, nout):
  return pl.pallas_call(
      _mid_body,
      grid=(GRID,),
      in_specs=[_row_spec(1), _vec_spec(D_H), _vec_spec(D_H), _vec_spec(D_H),
                _full_spec(D_H, W.shape[1])] + [_row_spec(128)] * 8,
      out_specs=[_row_spec(128)] * nout,
      out_shape=[jax.ShapeDtypeStruct((N, 128), jnp.float32)] * nout,
  )(dinv, b, g, be, W, *aggs, *us)


def _final_body(dinv_ref, b3_ref, x_ref, a0, a1, u0, u1, out_ref):
  agg = jnp.concatenate([a0[...], a1[...]], axis=1)
  u = jnp.concatenate([u0[...], u1[...]], axis=1)
  out_ref[...] = (agg + u) * dinv_ref[...] + b3_ref[...] + x_ref[...]


def _final(dinv, b3, x, aggs, us):
  return pl.pallas_call(
      _final_body,
      grid=(GRID,),
      in_specs=[_row_spec(1), _vec_spec(D_IN), _row_spec(D_IN)]
               + [_row_spec(128)] * 4,
      out_specs=_row_spec(D_IN),
      out_shape=jax.ShapeDtypeStruct((N, D_IN), jnp.float32),
  )(dinv, b3, x, *aggs, *us)


def kernel(x, edge_index, W0, b0, W1, b1, W2, b2, W3, b3,
           g0, be0, g1, be1, g2, be2):
  src = edge_index[0].reshape(NS, EPT)
  dst = edge_index[1].reshape(NS, EPT)
  npad = EPTP - EPT
  pad_src = jnp.broadcast_to((jnp.arange(npad, dtype=jnp.int32) * 521) % N,
                             (NS, npad))
  pad_dst = jnp.broadcast_to(N + (jnp.arange(npad, dtype=jnp.int32) % NS),
                             (NS, npad))
  srcp = jnp.concatenate([src, pad_src], axis=1).reshape(NS, NBLK, EB)
  dstp = jnp.concatenate([dst, pad_dst], axis=1).reshape(NS, NBLK, EB)

  ones_eb = jnp.ones((EB,), jnp.float32)
  zeros_deg = jnp.zeros((DEG_PT,), jnp.float32)
  zeros_acc = jnp.zeros((ZROWS, 128), jnp.float32)

  degp = _deg_kernel(dstp, ones_eb, zeros_deg).reshape(NC, DEG_P)
  dinv_row = _finalize_deg(degp)
  dinv = dinv_row.reshape(DEG_P, 1)[:N]

  ux = _prep(dinv, x)
  aggx = _agg2(srcp, dstp, zeros_acc, *ux)
  u1 = _layer0(dinv, W0, b0, g0, be0, W1, aggx, ux)
  agg1 = _agg4(srcp, dstp, zeros_acc, *u1)
  u2 = _mid_layer(dinv, b1, g1, be1, W2, agg1, u1, 4)
  agg2 = _agg4(srcp, dstp, zeros_acc, *u2)
  u3 = _mid_layer(dinv, b2, g2, be2, W3, agg2, u2, 2)
  agg3 = _agg2(srcp, dstp, zeros_acc, *u3)
  return _final(dinv, b3, x, agg3, u3)
